# Initial kernel scaffold; baseline (speedup 1.0000x reference)
import jax
import jax.numpy as jnp
from jax.experimental import pallas as pl


def kernel(x, edge_index, W1, b1, W2, b2, gn1_w, gn1_b, gn1_a, gn2_w, gn2_b, gn2_a):
    raise NotImplementedError("write your pallas kernel here")


# speedup vs baseline: 6.2973x; 6.2973x over previous
import functools

import jax
import jax.numpy as jnp
from jax import lax
from jax.experimental import pallas as pl
from jax.experimental.pallas import tpu as pltpu
from jax.experimental.pallas import tpu_sc as plsc

N = 10000
NP = 10240
E = 160000
EP = 163840
D = 256
H = 256
HH = H // 2
NC, NS = 2, 16
CW = 128
EPT = EP // NS
CH = EPT // CW
CHD = EPT // NC // CW
RPT = NP // NS
RB = 1000
NBLK = N // RB


def _sc_mesh():
    return plsc.VectorSubcoreMesh(core_axis_name="c", subcore_axis_name="s")


@functools.partial(
    pl.kernel,
    out_type=jax.ShapeDtypeStruct((NC, NP, 128), jnp.float32),
    mesh=_sc_mesh(),
    scratch_types=[
        pltpu.VMEM((CHD, CW), jnp.int32),
        pltpu.VMEM((CW,), jnp.int32),
        pltpu.VMEM((CW, 128), jnp.float32),
        pltpu.VMEM_SHARED((NP, 128), jnp.float32),
    ],
)
def _deg_kernel(dst_hbm, zeros_hbm, ones_hbm, out_hbm, dst_v, dst_cur, ones_v, acc):
    c = lax.axis_index("c")
    s = lax.axis_index("s")
    pltpu.sync_copy(dst_hbm.at[s, pl.ds(c * CHD, CHD)], dst_v)
    pltpu.sync_copy(ones_hbm, ones_v)
    pltpu.sync_copy(zeros_hbm.at[pl.ds(s * RPT, RPT)], acc.at[pl.ds(s * RPT, RPT)])
    plsc.subcore_barrier()

    def body(k, carry):
        for i in range(CW // 16):
            dst_cur[pl.ds(i * 16, 16)] = dst_v[k, pl.ds(i * 16, 16)]
        pltpu.sync_copy(ones_v, acc.at[dst_cur], add=True)
        return carry

    lax.fori_loop(0, CHD, body, 0)
    plsc.subcore_barrier()
    pltpu.sync_copy(acc.at[pl.ds(s * RPT, RPT)], out_hbm.at[c, pl.ds(s * RPT, RPT)])


@functools.partial(
    pl.kernel,
    out_type=jax.ShapeDtypeStruct((NC, NP, HH), jnp.float32),
    mesh=_sc_mesh(),
    scratch_types=[
        pltpu.VMEM((CH, CW), jnp.int32),
        pltpu.VMEM((CH, CW), jnp.int32),
        pltpu.VMEM((CW,), jnp.int32),
        pltpu.VMEM((CW,), jnp.int32),
        pltpu.VMEM((CW, HH), jnp.float32),
        pltpu.VMEM_SHARED((NP, HH), jnp.float32),
        pltpu.SemaphoreType.DMA,
    ],
)
def _conv_kernel(hs_hbm, src_hbm, dst_hbm, out_hbm, src_v, dst_v, src_cur, dst_cur,
                 gbuf, acc, sem):
    c = lax.axis_index("c")
    s = lax.axis_index("s")
    pltpu.sync_copy(src_hbm.at[s], src_v)
    pltpu.sync_copy(dst_hbm.at[s], dst_v)
    pltpu.sync_copy(hs_hbm.at[c, pl.ds(s * RPT, RPT)], acc.at[pl.ds(s * RPT, RPT)])
    plsc.subcore_barrier()
    table = hs_hbm.at[c]

    def body(k, carry):
        for i in range(CW // 16):
            src_cur[pl.ds(i * 16, 16)] = src_v[k, pl.ds(i * 16, 16)]
            dst_cur[pl.ds(i * 16, 16)] = dst_v[k, pl.ds(i * 16, 16)]
        pltpu.async_copy(table.at[src_cur], gbuf, sem).wait()
        pltpu.sync_copy(gbuf, acc.at[dst_cur], add=True)
        return carry

    lax.fori_loop(0, CH, body, 0)
    plsc.subcore_barrier()
    pltpu.sync_copy(acc.at[pl.ds(s * RPT, RPT)], out_hbm.at[c, pl.ds(s * RPT, RPT)])


def _dinv_block(degp):
    deg = degp[0, :, 0:1] + degp[1, :, 0:1] + 1.0
    return lax.rsqrt(deg)


def _pre_body(x_ref, w_ref, degp_ref, out_ref):
    h = jnp.dot(x_ref[...], w_ref[...], preferred_element_type=jnp.float32)
    hs = h * _dinv_block(degp_ref[...])
    out_ref[0, :, :] = hs[:, :HH]
    out_ref[1, :, :] = hs[:, HH:]


def _tc_pre(x, W1, degp):
    return pl.pallas_call(
        _pre_body,
        grid=(NBLK,),
        in_specs=[
            pl.BlockSpec((RB, D), lambda i: (i, 0)),
            pl.BlockSpec((D, H), lambda i: (0, 0)),
            pl.BlockSpec((2, RB, 128), lambda i: (0, i, 0)),
        ],
        out_specs=pl.BlockSpec((2, RB, HH), lambda i: (0, i, 0)),
        out_shape=jax.ShapeDtypeStruct((2, NP, HH), jnp.float32),
    )(x, W1, degp)


def _z_block(agg_ref, degp_ref, b_ref):
    agg = jnp.concatenate([agg_ref[0], agg_ref[1]], axis=1)
    return agg * _dinv_block(degp_ref[...]) + b_ref[...]


def _gn_coeffs(s1_ref, s2_ref, w_ref, b_ref, a_ref, eps=1e-5):
    m = s1_ref[...] / N
    var = s2_ref[...] / N - m * m * a_ref[...] * (2.0 - a_ref[...])
    cmul = w_ref[...] * lax.rsqrt(var + eps)
    cadd = b_ref[...] - a_ref[...] * m * cmul
    return cmul, cadd


def _mid_body(agg_ref, degp_ref, b1_ref, w_ref, bb_ref, a_ref, w2_ref,
              out_ref, s1, s2, cmul, cadd):
    j = pl.program_id(0)
    i = pl.program_id(1)

    @pl.when(jnp.logical_and(j == 0, i == 0))
    def _():
        s1[...] = jnp.zeros_like(s1)
        s2[...] = jnp.zeros_like(s2)

    z = _z_block(agg_ref, degp_ref, b1_ref)

    @pl.when(j == 0)
    def _():
        s1[...] += jnp.sum(z, axis=0, keepdims=True)
        s2[...] += jnp.sum(z * z, axis=0, keepdims=True)

    @pl.when(jnp.logical_and(j == 1, i == 0))
    def _():
        cm, ca = _gn_coeffs(s1, s2, w_ref, bb_ref, a_ref)
        cmul[...] = cm
        cadd[...] = ca

    @pl.when(j == 1)
    def _():
        g = jnp.maximum(z * cmul[...] + cadd[...], 0.0)
        h2 = jnp.dot(g, w2_ref[...], preferred_element_type=jnp.float32)
        hs = h2 * _dinv_block(degp_ref[...])
        out_ref[0, :, :] = hs[:, :HH]
        out_ref[1, :, :] = hs[:, HH:]


def _tc_mid(agg1, degp, b1, gn1_w, gn1_b, gn1_a, W2):
    return pl.pallas_call(
        _mid_body,
        grid=(2, NBLK),
        in_specs=[
            pl.BlockSpec((2, RB, HH), lambda j, i: (0, i, 0)),
            pl.BlockSpec((2, RB, 128), lambda j, i: (0, i, 0)),
            pl.BlockSpec((1, H), lambda j, i: (0, 0)),
            pl.BlockSpec((1, H), lambda j, i: (0, 0)),
            pl.BlockSpec((1, H), lambda j, i: (0, 0)),
            pl.BlockSpec((1, H), lambda j, i: (0, 0)),
            pl.BlockSpec((H, H), lambda j, i: (0, 0)),
        ],
        out_specs=pl.BlockSpec((2, RB, HH), lambda j, i: (0, i, 0)),
        out_shape=jax.ShapeDtypeStruct((2, NP, HH), jnp.float32),
        scratch_shapes=[
            pltpu.VMEM((1, H), jnp.float32),
            pltpu.VMEM((1, H), jnp.float32),
            pltpu.VMEM((1, H), jnp.float32),
            pltpu.VMEM((1, H), jnp.float32),
        ],
        compiler_params=pltpu.CompilerParams(
            dimension_semantics=("arbitrary", "arbitrary")),
    )(agg1, degp, b1.reshape(1, H), gn1_w.reshape(1, H),
      gn1_b.reshape(1, H), gn1_a.reshape(1, H), W2)


def _fin_body(agg_ref, degp_ref, b2_ref, w_ref, bb_ref, a_ref, x_ref,
              out_ref, s1, s2, cmul, cadd):
    j = pl.program_id(0)
    i = pl.program_id(1)

    @pl.when(jnp.logical_and(j == 0, i == 0))
    def _():
        s1[...] = jnp.zeros_like(s1)
        s2[...] = jnp.zeros_like(s2)

    z = _z_block(agg_ref, degp_ref, b2_ref)

    @pl.when(j == 0)
    def _():
        s1[...] += jnp.sum(z, axis=0, keepdims=True)
        s2[...] += jnp.sum(z * z, axis=0, keepdims=True)

    @pl.when(jnp.logical_and(j == 1, i == 0))
    def _():
        cm, ca = _gn_coeffs(s1, s2, w_ref, bb_ref, a_ref)
        cmul[...] = cm
        cadd[...] = ca

    @pl.when(j == 1)
    def _():
        out_ref[...] = jnp.maximum(z * cmul[...] + cadd[...] + x_ref[...], 0.0)


def _tc_fin(agg2, degp, b2, gn2_w, gn2_b, gn2_a, x):
    return pl.pallas_call(
        _fin_body,
        grid=(2, NBLK),
        in_specs=[
            pl.BlockSpec((2, RB, HH), lambda j, i: (0, i, 0)),
            pl.BlockSpec((2, RB, 128), lambda j, i: (0, i, 0)),
            pl.BlockSpec((1, H), lambda j, i: (0, 0)),
            pl.BlockSpec((1, H), lambda j, i: (0, 0)),
            pl.BlockSpec((1, H), lambda j, i: (0, 0)),
            pl.BlockSpec((1, H), lambda j, i: (0, 0)),
            pl.BlockSpec((RB, H), lambda j, i: (i, 0)),
        ],
        out_specs=pl.BlockSpec((RB, H), lambda j, i: (i, 0)),
        out_shape=jax.ShapeDtypeStruct((N, H), jnp.float32),
        scratch_shapes=[
            pltpu.VMEM((1, H), jnp.float32),
            pltpu.VMEM((1, H), jnp.float32),
            pltpu.VMEM((1, H), jnp.float32),
            pltpu.VMEM((1, H), jnp.float32),
        ],
        compiler_params=pltpu.CompilerParams(
            dimension_semantics=("arbitrary", "arbitrary")),
    )(agg2, degp, b2.reshape(1, H), gn2_w.reshape(1, H),
      gn2_b.reshape(1, H), gn2_a.reshape(1, H), x)


def _conv_jnp(hs, src, dst):
    return hs.at[:, dst, :].add(hs[:, src, :])


def kernel(x, edge_index, W1, b1, W2, b2, gn1_w, gn1_b, gn1_a, gn2_w, gn2_b, gn2_a):
    pad = jnp.full((2, EP - E), N, dtype=edge_index.dtype)
    ei = jnp.concatenate([edge_index, pad], axis=1)
    src = ei[0].reshape(NS, CH, CW)
    dst = ei[1].reshape(NS, CH, CW)
    zeros16 = jnp.zeros((NP, 128), jnp.float32)
    ones16 = jnp.ones((CW, 128), jnp.float32)

    degp = _deg_kernel(dst, zeros16, ones16)
    hs1 = _tc_pre(x, W1, degp)
    agg1 = _conv_kernel(hs1, src, dst)
    hs2 = _tc_mid(agg1, degp, b1, gn1_w, gn1_b, gn1_a, W2)
    agg2 = _conv_kernel(hs2, src, dst)
    return _tc_fin(agg2, degp, b2, gn2_w, gn2_b, gn2_a, x)
